# Initial kernel scaffold; baseline (speedup 1.0000x reference)
import jax
import jax.numpy as jnp
from jax.experimental import pallas as pl


def kernel(x, edge_index, W, b):
    raise NotImplementedError("write your pallas kernel here")


# speedup vs baseline: 17.5040x; 17.5040x over previous
import functools

import jax
import jax.numpy as jnp
from jax import lax
from jax.experimental import pallas as pl
from jax.experimental.pallas import tpu as pltpu
from jax.experimental.pallas import tpu_sc as plsc

N_NODES = 10000
N_PAD = 10240
N_EDGES = 320000
D = 128
NC = 2
NS = 16
NW = NC * NS
E_PER_W = N_EDGES // NW
K = 80
ROWS_PER_TILE = N_PAD // NS

_mesh = plsc.VectorSubcoreMesh(core_axis_name="c", subcore_axis_name="s")


@functools.partial(
    pl.kernel,
    out_type=jax.ShapeDtypeStruct((NC, N_PAD), jnp.float32),
    mesh=_mesh,
    scratch_types=[
        pltpu.VMEM((K,), jnp.int32),
        pltpu.VMEM((K,), jnp.float32),
        pltpu.VMEM((ROWS_PER_TILE,), jnp.float32),
        pltpu.VMEM_SHARED((N_PAD,), jnp.float32),
    ],
)
def _deg_kernel(row_hbm, out_hbm, idx_v, ones_v, z_v, bins_sh):
    cid = lax.axis_index("c")
    sid = lax.axis_index("s")
    wid = sid * NC + cid
    for i in range(K // 16):
        ones_v[pl.ds(i * 16, 16)] = jnp.ones((16,), jnp.float32)

    def zbody(i, _):
        z_v[pl.ds(i * 16, 16)] = jnp.zeros((16,), jnp.float32)
        return 0

    lax.fori_loop(0, ROWS_PER_TILE // 16, zbody, 0)
    tile_sl = pl.ds(sid * ROWS_PER_TILE, ROWS_PER_TILE)
    pltpu.sync_copy(z_v, bins_sh.at[tile_sl])
    plsc.subcore_barrier()

    base = wid * E_PER_W

    def body(j, _):
        pltpu.sync_copy(row_hbm.at[pl.ds(base + j * K, K)], idx_v)
        pltpu.sync_copy(ones_v, bins_sh.at[idx_v], add=True)
        return 0

    lax.fori_loop(0, E_PER_W // K, body, 0)
    plsc.subcore_barrier()
    pltpu.sync_copy(bins_sh.at[tile_sl], out_hbm.at[cid, tile_sl])


@functools.partial(
    pl.kernel,
    out_type=jax.ShapeDtypeStruct((NC, N_PAD, D), jnp.float32),
    mesh=_mesh,
    scratch_types=[
        pltpu.VMEM((K,), jnp.int32),
        pltpu.VMEM((K,), jnp.int32),
        pltpu.VMEM((K, D), jnp.float32),
        pltpu.VMEM_SHARED((N_PAD, D), jnp.float32),
        pltpu.SemaphoreType.DMA,
    ],
)
def _scatter_kernel(g_hbm, col_hbm, row_hbm, out_hbm,
                    col_v, row_v, rows_v, acc_sh, sem):
    cid = lax.axis_index("c")
    sid = lax.axis_index("s")
    wid = sid * NC + cid

    def zbody(i, _):
        r = i // (D // 16)
        c = lax.rem(i, D // 16)
        rows_v[r, pl.ds(c * 16, 16)] = jnp.zeros((16,), jnp.float32)
        return 0

    lax.fori_loop(0, K * (D // 16), zbody, 0)
    for i in range(ROWS_PER_TILE // K):
        pltpu.sync_copy(
            rows_v, acc_sh.at[pl.ds(sid * ROWS_PER_TILE + i * K, K), :])
    plsc.subcore_barrier()

    base = wid * E_PER_W

    def body(j, _):
        e0 = base + j * K
        pltpu.sync_copy(col_hbm.at[pl.ds(e0, K)], col_v)
        pltpu.sync_copy(row_hbm.at[pl.ds(e0, K)], row_v)
        pltpu.async_copy(g_hbm.at[col_v], rows_v, sem).wait()
        pltpu.sync_copy(rows_v, acc_sh.at[row_v], add=True)
        return 0

    lax.fori_loop(0, E_PER_W // K, body, 0)
    plsc.subcore_barrier()
    tile_sl = pl.ds(sid * ROWS_PER_TILE, ROWS_PER_TILE)
    pltpu.sync_copy(acc_sh.at[tile_sl, :], out_hbm.at[cid, tile_sl, :])


BR = 512


@functools.partial(
    pl.pallas_call,
    out_shape=jax.ShapeDtypeStruct((N_PAD, D), jnp.float32),
    grid=(N_PAD // BR,),
    in_specs=[
        pl.BlockSpec((BR, D), lambda i: (i, 0)),
        pl.BlockSpec((D, D), lambda i: (0, 0)),
        pl.BlockSpec((BR, NC), lambda i: (i, 0)),
    ],
    out_specs=pl.BlockSpec((BR, D), lambda i: (i, 0)),
)
def _matmul_norm(x_ref, w_ref, deg_ref, g_ref):
    deg = deg_ref[:, 0:1] + deg_ref[:, 1:2] + 1.0
    dinv = lax.rsqrt(deg)
    h = jnp.dot(x_ref[...], w_ref[...], preferred_element_type=jnp.float32)
    g_ref[...] = dinv * h


@functools.partial(
    pl.pallas_call,
    out_shape=jax.ShapeDtypeStruct((N_PAD, D), jnp.float32),
    grid=(N_PAD // BR,),
    in_specs=[
        pl.BlockSpec((NC, BR, D), lambda i: (0, i, 0)),
        pl.BlockSpec((BR, D), lambda i: (i, 0)),
        pl.BlockSpec((BR, NC), lambda i: (i, 0)),
        pl.BlockSpec((1, D), lambda i: (0, 0)),
    ],
    out_specs=pl.BlockSpec((BR, D), lambda i: (i, 0)),
)
def _final_combine(acc_ref, g_ref, deg_ref, b_ref, out_ref):
    deg = deg_ref[:, 0:1] + deg_ref[:, 1:2] + 1.0
    dinv = lax.rsqrt(deg)
    out_ref[...] = dinv * (acc_ref[0] + acc_ref[1] + g_ref[...]) + b_ref[...]


def kernel(x, edge_index, W, b):
    row = edge_index[0]
    col = edge_index[1]
    x_pad = jnp.pad(x, ((0, N_PAD - N_NODES), (0, 0)))
    degp = _deg_kernel(row)
    degp_t = degp.T
    g = _matmul_norm(x_pad, W, degp_t)
    acc = _scatter_kernel(g, col, row)
    out = _final_combine(acc, g, degp_t, b.reshape(1, D))
    return out[:N_NODES]


# speedup vs baseline: 42.0643x; 2.4031x over previous
import functools

import jax
import jax.numpy as jnp
from jax import lax
from jax.experimental import pallas as pl
from jax.experimental.pallas import tpu as pltpu
from jax.experimental.pallas import tpu_sc as plsc

N_NODES = 10000
N_PAD = 10240
N_EDGES = 320000
D = 128
NC = 2
NS = 16
NW = NC * NS
E_PER_W = N_EDGES // NW
K = 125
NCHUNK = E_PER_W // K
NBUF = 2
ROWS_PER_TILE = N_PAD // NS

_mesh = plsc.VectorSubcoreMesh(core_axis_name="c", subcore_axis_name="s")


@functools.partial(
    pl.kernel,
    out_type=jax.ShapeDtypeStruct((NC, N_PAD), jnp.float32),
    mesh=_mesh,
    scratch_types=[
        pltpu.VMEM((NCHUNK, K), jnp.int32),
        pltpu.VMEM((K,), jnp.float32),
        pltpu.VMEM((ROWS_PER_TILE,), jnp.float32),
        pltpu.VMEM_SHARED((N_PAD,), jnp.float32),
        pltpu.SemaphoreType.DMA,
    ],
)
def _deg_kernel(row_hbm, out_hbm, idx_v, ones_v, z_v, bins_sh, sem):
    cid = lax.axis_index("c")
    sid = lax.axis_index("s")
    wid = sid * NC + cid
    for i in range(K // 16 + 1):
        o = min(i * 16, K - 16)
        ones_v[pl.ds(o, 16)] = jnp.ones((16,), jnp.float32)

    def zbody(i, _):
        z_v[pl.ds(i * 16, 16)] = jnp.zeros((16,), jnp.float32)
        return 0

    lax.fori_loop(0, ROWS_PER_TILE // 16, zbody, 0)
    tile_sl = pl.ds(sid * ROWS_PER_TILE, ROWS_PER_TILE)
    pltpu.sync_copy(z_v, bins_sh.at[tile_sl])
    pltpu.sync_copy(row_hbm.at[wid], idx_v)
    plsc.subcore_barrier()

    GRP = 8

    def body(j, _):
        descs = []
        for b in range(GRP):
            descs.append(pltpu.async_copy(
                ones_v, bins_sh.at[idx_v.at[j * GRP + b]], sem, add=True))
        for d in descs:
            d.wait()
        return 0

    lax.fori_loop(0, NCHUNK // GRP, body, 0)
    plsc.subcore_barrier()
    pltpu.sync_copy(bins_sh.at[tile_sl], out_hbm.at[cid, tile_sl])


ZR = 80


@functools.partial(
    pl.kernel,
    out_type=jax.ShapeDtypeStruct((NC, N_PAD, D), jnp.float32),
    mesh=_mesh,
    scratch_types=[
        pltpu.VMEM((NCHUNK, K), jnp.int32),
        [pltpu.VMEM((K,), jnp.int32)] * NBUF,
        [pltpu.VMEM((K, D), jnp.float32)] * NBUF,
        pltpu.VMEM_SHARED((N_PAD, D), jnp.float32),
        pltpu.SemaphoreType.DMA,
        pltpu.SemaphoreType.DMA,
    ],
)
def _scatter_kernel(g_hbm, col_hbm, row_hbm, out_hbm,
                    col_v, rowb, bufs, acc_sh, gsem, rsem):
    cid = lax.axis_index("c")
    sid = lax.axis_index("s")
    wid = sid * NC + cid

    def zbody(i, _):
        r = i // (D // 16)
        c = lax.rem(i, D // 16)
        bufs[0][r, pl.ds(c * 16, 16)] = jnp.zeros((16,), jnp.float32)
        return 0

    lax.fori_loop(0, ZR * (D // 16), zbody, 0)
    zsrc = bufs[0].at[pl.ds(0, ZR), :]
    for i in range(ROWS_PER_TILE // ZR):
        pltpu.sync_copy(
            zsrc, acc_sh.at[pl.ds(sid * ROWS_PER_TILE + i * ZR, ZR), :])
    pltpu.sync_copy(col_hbm.at[wid], col_v)
    for b in range(NBUF):
        pltpu.async_copy(row_hbm.at[wid, b], rowb[b], rsem)
        pltpu.async_copy(g_hbm.at[col_v.at[b]], bufs[b], gsem)
    plsc.subcore_barrier()

    def body(j, _):
        for b in range(NBUF):
            jj = j * NBUF + b
            pltpu.make_async_copy(row_hbm.at[wid, jj], rowb[b], rsem).wait()
            pltpu.make_async_copy(g_hbm.at[col_v.at[jj]], bufs[b], gsem).wait()
            pltpu.sync_copy(bufs[b], acc_sh.at[rowb[b]], add=True)

            @pl.when(jj + NBUF < NCHUNK)
            def _():
                pltpu.async_copy(row_hbm.at[wid, jj + NBUF], rowb[b], rsem)
                pltpu.async_copy(g_hbm.at[col_v.at[jj + NBUF]], bufs[b], gsem)

        return 0

    lax.fori_loop(0, NCHUNK // NBUF, body, 0)
    plsc.subcore_barrier()
    tile_sl = pl.ds(sid * ROWS_PER_TILE, ROWS_PER_TILE)
    pltpu.sync_copy(acc_sh.at[tile_sl, :], out_hbm.at[cid, tile_sl, :])


BR = 512


@functools.partial(
    pl.pallas_call,
    out_shape=jax.ShapeDtypeStruct((N_PAD, D), jnp.float32),
    grid=(N_PAD // BR,),
    in_specs=[
        pl.BlockSpec((BR, D), lambda i: (i, 0)),
        pl.BlockSpec((D, D), lambda i: (0, 0)),
        pl.BlockSpec((BR, NC), lambda i: (i, 0)),
    ],
    out_specs=pl.BlockSpec((BR, D), lambda i: (i, 0)),
)
def _matmul_norm(x_ref, w_ref, deg_ref, g_ref):
    deg = deg_ref[:, 0:1] + deg_ref[:, 1:2] + 1.0
    dinv = lax.rsqrt(deg)
    h = jnp.dot(x_ref[...], w_ref[...], preferred_element_type=jnp.float32)
    g_ref[...] = dinv * h


@functools.partial(
    pl.pallas_call,
    out_shape=jax.ShapeDtypeStruct((N_PAD, D), jnp.float32),
    grid=(N_PAD // BR,),
    in_specs=[
        pl.BlockSpec((NC, BR, D), lambda i: (0, i, 0)),
        pl.BlockSpec((BR, D), lambda i: (i, 0)),
        pl.BlockSpec((BR, NC), lambda i: (i, 0)),
        pl.BlockSpec((1, D), lambda i: (0, 0)),
    ],
    out_specs=pl.BlockSpec((BR, D), lambda i: (i, 0)),
)
def _final_combine(acc_ref, g_ref, deg_ref, b_ref, out_ref):
    deg = deg_ref[:, 0:1] + deg_ref[:, 1:2] + 1.0
    dinv = lax.rsqrt(deg)
    out_ref[...] = dinv * (acc_ref[0] + acc_ref[1] + g_ref[...]) + b_ref[...]


def kernel(x, edge_index, W, b):
    row = edge_index[0].reshape(NW, NCHUNK, K)
    col = edge_index[1].reshape(NW, NCHUNK, K)
    x_pad = jnp.pad(x, ((0, N_PAD - N_NODES), (0, 0)))
    degp = _deg_kernel(row)
    degp_t = degp.T
    g = _matmul_norm(x_pad, W, degp_t)
    acc = _scatter_kernel(g, col, row)
    out = _final_combine(acc, g, degp_t, b.reshape(1, D))
    return out[:N_NODES]
